# baseline (speedup 1.0000x reference)
import functools

import numpy as np
import jax
import jax.numpy as jnp
from jax import lax
from jax.experimental import pallas as pl
from jax.experimental.pallas import tpu as pltpu, tpu_sc as plsc

_D_MODEL = 1024
_MAX_LEN = 8192


def _pos_encoding_np(max_len, d_model):
    pos = np.arange(max_len, dtype=np.float32)[:, None]
    i = np.arange(0, d_model, 2, dtype=np.float32)
    div = np.power(10000.0, i / d_model)
    enc = np.zeros((max_len, d_model), dtype=np.float32)
    enc[:, 0::2] = np.sin(pos / div)
    enc[:, 1::2] = np.cos(pos / div)
    return enc


_PE_NP = _pos_encoding_np(_MAX_LEN, _D_MODEL)


@functools.lru_cache(maxsize=None)
def _build(B, S, D, C):
    info = plsc.get_sparse_core_info()
    NW = info.num_cores * info.num_subcores
    P = S // NW
    n_chunks = P // C
    v16 = D // 16

    mesh = plsc.VectorSubcoreMesh(core_axis_name="c", subcore_axis_name="s")

    @functools.partial(
        pl.kernel,
        mesh=mesh,
        out_type=jax.ShapeDtypeStruct((B * S, D), jnp.float32),
        scratch_types=[
            pltpu.VMEM((B * P,), jnp.int32),
            pltpu.VMEM((2, B * C, D), jnp.float32),
            pltpu.VMEM((2, C, D), jnp.float32),
        ] + [pltpu.SemaphoreType.DMA] * 12,
    )
    def k(idx_hbm, table_hbm, pe_hbm, out_hbm, idx_v, rows_v, pe_v,
          sp0, sp1, sa0, sa1, sa2, sa3, sb0, sb1, sb2, sb3, so0, so1):
        wid = lax.axis_index("s") * info.num_cores + lax.axis_index("c")
        p0 = wid * P
        sp = (sp0, sp1)
        sgb = ((sa0, sa1, sa2, sa3), (sb0, sb1, sb2, sb3))
        so = (so0, so1)
        pltpu.sync_copy(idx_hbm.at[pl.ds(wid * B * P, B * P)], idx_v)

        def issue_gathers(ci, buf):
            pltpu.async_copy(pe_hbm.at[pl.ds(p0 + ci * C, C)],
                             pe_v.at[buf], sp[buf])
            for b in range(B):
                pltpu.async_copy(
                    table_hbm.at[idx_v.at[pl.ds(ci * B * C + b * C, C)]],
                    rows_v.at[buf, pl.ds(b * C, C)],
                    sgb[buf][b],
                )

        def wait_outs(buf):
            for b in range(B):
                pltpu.make_async_copy(
                    rows_v.at[buf, pl.ds(b * C, C)],
                    out_hbm.at[pl.ds(b * C, C)],
                    so[buf],
                ).wait()

        issue_gathers(0, 0)

        def pair_body(g, _):
            for buf in (0, 1):
                nb = 1 - buf
                ci = 2 * g + buf

                @pl.when(ci >= 1)
                def _():
                    wait_outs(nb)

                @pl.when(ci + 1 < n_chunks)
                def _():
                    issue_gathers(ci + 1, nb)

                pltpu.make_async_copy(
                    pe_hbm.at[pl.ds(0, C)], pe_v.at[buf], sp[buf]
                ).wait()
                for b in range(B):
                    pltpu.make_async_copy(
                        table_hbm.at[idx_v.at[pl.ds(0, C)]],
                        rows_v.at[buf, pl.ds(b * C, C)],
                        sgb[buf][b],
                    ).wait()

                    def add_body(r, _, b=b, buf=buf):
                        def col_body(j, _):
                            col = j * 16
                            pe_vec = pe_v[buf, r, pl.ds(col, 16)]
                            plsc.addupdate(
                                rows_v.at[buf, b * C + r, pl.ds(col, 16)],
                                pe_vec,
                            )
                            return 0

                        lax.fori_loop(0, v16, col_body, 0, unroll=4)
                        return 0

                    lax.fori_loop(0, C, add_body, 0)
                    pltpu.async_copy(
                        rows_v.at[buf, pl.ds(b * C, C)],
                        out_hbm.at[pl.ds(b * S + p0 + ci * C, C)],
                        so[buf],
                    )
            return 0

        lax.fori_loop(0, n_chunks // 2, pair_body, 0)
        wait_outs(1)

    return k


def kernel(x, tok_table):
    B, S = x.shape
    V, D = tok_table.shape
    C = 8
    NW = 32
    n_chunks = S // NW // C
    idx = (
        x.astype(jnp.int32)
        .reshape(B, NW, n_chunks, C)
        .transpose(1, 2, 0, 3)
        .reshape(-1)
    )
    pe = jnp.asarray(_PE_NP[:S], dtype=jnp.float32)
    out = _build(B, S, D, C)(idx, tok_table, pe)
    return out.reshape(B, S, D)


# speedup vs baseline: 1.7259x; 1.7259x over previous
import functools

import numpy as np
import jax
import jax.numpy as jnp
from jax import lax
from jax.experimental import pallas as pl
from jax.experimental.pallas import tpu as pltpu, tpu_sc as plsc

_D_MODEL = 1024
_MAX_LEN = 8192


def _pos_encoding_np(max_len, d_model):
    pos = np.arange(max_len, dtype=np.float32)[:, None]
    i = np.arange(0, d_model, 2, dtype=np.float32)
    div = np.power(10000.0, i / d_model)
    enc = np.zeros((max_len, d_model), dtype=np.float32)
    enc[:, 0::2] = np.sin(pos / div)
    enc[:, 1::2] = np.cos(pos / div)
    return enc


_PE_NP = _pos_encoding_np(_MAX_LEN, _D_MODEL)


@functools.lru_cache(maxsize=None)
def _build(B, S, D, C):
    info = plsc.get_sparse_core_info()
    NW = info.num_cores * info.num_subcores
    P = S // NW
    n_chunks = P // C
    v16 = D // 16

    mesh = plsc.VectorSubcoreMesh(core_axis_name="c", subcore_axis_name="s")

    NBUF = 3

    @functools.partial(
        pl.kernel,
        mesh=mesh,
        out_type=jax.ShapeDtypeStruct((B * S, D), jnp.float32),
        scratch_types=[
            pltpu.VMEM((B * P,), jnp.int32),
            pltpu.VMEM((NBUF, B * C, D), jnp.float32),
            pltpu.VMEM((NBUF, C, D), jnp.float32),
        ] + [pltpu.SemaphoreType.DMA] * (2 * NBUF),
    )
    def k(idx_hbm, table_hbm, pe_hbm, out_hbm, idx_v, rows_v, pe_v,
          sg0, sg1, sg2, so0, so1, so2):
        sg = (sg0, sg1, sg2)
        so = (so0, so1, so2)
        wid = lax.axis_index("s") * info.num_cores + lax.axis_index("c")
        p0 = wid * P
        pltpu.sync_copy(idx_hbm.at[pl.ds(wid * B * P, B * P)], idx_v)

        def issue_gathers(ci, buf):
            cps = [
                pltpu.async_copy(
                    table_hbm.at[idx_v.at[pl.ds(ci * B * C, B * C)]],
                    rows_v.at[buf],
                    sg[buf],
                ),
                pltpu.async_copy(pe_hbm.at[pl.ds(p0 + ci * C, C)],
                                 pe_v.at[buf], sg[buf]),
            ]
            return cps

        out_cps = {b: [] for b in range(NBUF)}
        gat_cps = {}
        gat_cps[0] = issue_gathers(0, 0)
        gat_cps[1] = issue_gathers(1, 1)

        for ci in range(n_chunks):
            buf = ci % NBUF
            for cp in gat_cps[buf]:
                cp.wait()

            def add_body(r, _, buf=buf):
                def col_body(j, _):
                    col = j * 16
                    pe_vec = pe_v[buf, r, pl.ds(col, 16)]
                    for b in range(B):
                        plsc.addupdate(
                            rows_v.at[buf, b * C + r, pl.ds(col, 16)], pe_vec
                        )
                    return 0

                lax.fori_loop(0, v16, col_body, 0, unroll=4)
                return 0

            lax.fori_loop(0, C, add_body, 0)
            off = ci * C
            out_cps[buf] = [
                pltpu.async_copy(
                    rows_v.at[buf, pl.ds(b * C, C)],
                    out_hbm.at[pl.ds(b * S + p0 + off, C)],
                    so[buf],
                )
                for b in range(B)
            ]
            if ci + 1 < n_chunks:
                nb = (ci + 2) % NBUF
                for cp in out_cps[nb]:
                    cp.wait()
                out_cps[nb] = []
                if ci + 2 < n_chunks:
                    gat_cps[nb] = issue_gathers(ci + 2, nb)
        for b in range(NBUF):
            for cp in out_cps[b]:
                cp.wait()

    return k


def kernel(x, tok_table):
    B, S = x.shape
    V, D = tok_table.shape
    C = 8
    NW = 32
    n_chunks = S // NW // C
    idx = (
        x.astype(jnp.int32)
        .reshape(B, NW, n_chunks, C)
        .transpose(1, 2, 0, 3)
        .reshape(-1)
    )
    pe = jnp.asarray(_PE_NP[:S], dtype=jnp.float32)
    out = _build(B, S, D, C)(idx, tok_table, pe)
    return out.reshape(B, S, D)


# speedup vs baseline: 1.7377x; 1.0068x over previous
import functools

import numpy as np
import jax
import jax.numpy as jnp
from jax import lax
from jax.experimental import pallas as pl
from jax.experimental.pallas import tpu as pltpu, tpu_sc as plsc

_D_MODEL = 1024
_MAX_LEN = 8192


def _pos_encoding_np(max_len, d_model):
    pos = np.arange(max_len, dtype=np.float32)[:, None]
    i = np.arange(0, d_model, 2, dtype=np.float32)
    div = np.power(10000.0, i / d_model)
    enc = np.zeros((max_len, d_model), dtype=np.float32)
    enc[:, 0::2] = np.sin(pos / div)
    enc[:, 1::2] = np.cos(pos / div)
    return enc


_PE_NP = _pos_encoding_np(_MAX_LEN, _D_MODEL)


@functools.lru_cache(maxsize=None)
def _build(B, S, D, C):
    info = plsc.get_sparse_core_info()
    NW = info.num_cores * info.num_subcores
    P = S // NW
    n_chunks = P // C
    v16 = D // 16

    mesh = plsc.VectorSubcoreMesh(core_axis_name="c", subcore_axis_name="s")

    NBUF = 3

    @functools.partial(
        pl.kernel,
        mesh=mesh,
        out_type=jax.ShapeDtypeStruct((B * S, D), jnp.float32),
        scratch_types=[
            pltpu.VMEM((B * P,), jnp.int32),
            pltpu.VMEM((NBUF, B * C, D), jnp.float32),
            pltpu.VMEM((NBUF, C, D), jnp.float32),
        ] + [pltpu.SemaphoreType.DMA] * (2 * NBUF),
    )
    def k(idx_hbm, table_hbm, pe_hbm, out_hbm, idx_v, rows_v, pe_v,
          sg0, sg1, sg2, so0, so1, so2):
        sg = (sg0, sg1, sg2)
        so = (so0, so1, so2)
        wid = lax.axis_index("s") * info.num_cores + lax.axis_index("c")
        p0 = wid * P
        for b in range(B):
            pltpu.sync_copy(
                idx_hbm.at[pl.ds(b * S + p0, P)], idx_v.at[pl.ds(b * P, P)]
            )

        def issue_gathers(ci, buf):
            cps = [
                pltpu.async_copy(
                    table_hbm.at[idx_v.at[pl.ds(b * P + ci * C, C)]],
                    rows_v.at[buf, pl.ds(b * C, C)],
                    sg[buf],
                )
                for b in range(B)
            ]
            cps.append(
                pltpu.async_copy(pe_hbm.at[pl.ds(p0 + ci * C, C)],
                                 pe_v.at[buf], sg[buf])
            )
            return cps

        out_cps = {b: [] for b in range(NBUF)}
        gat_cps = {}
        gat_cps[0] = issue_gathers(0, 0)
        gat_cps[1] = issue_gathers(1, 1)

        for ci in range(n_chunks):
            buf = ci % NBUF
            for cp in gat_cps[buf]:
                cp.wait()

            def add_body(r, _, buf=buf):
                def col_body(j, _):
                    col = j * 16
                    pe_vec = pe_v[buf, r, pl.ds(col, 16)]
                    for b in range(B):
                        plsc.addupdate(
                            rows_v.at[buf, b * C + r, pl.ds(col, 16)], pe_vec
                        )
                    return 0

                lax.fori_loop(0, v16, col_body, 0, unroll=4)
                return 0

            lax.fori_loop(0, C, add_body, 0)
            off = ci * C
            out_cps[buf] = [
                pltpu.async_copy(
                    rows_v.at[buf, pl.ds(b * C, C)],
                    out_hbm.at[pl.ds(b * S + p0 + off, C)],
                    so[buf],
                )
                for b in range(B)
            ]
            if ci + 1 < n_chunks:
                nb = (ci + 2) % NBUF
                for cp in out_cps[nb]:
                    cp.wait()
                out_cps[nb] = []
                if ci + 2 < n_chunks:
                    gat_cps[nb] = issue_gathers(ci + 2, nb)
        for b in range(NBUF):
            for cp in out_cps[b]:
                cp.wait()

    return k


def kernel(x, tok_table):
    B, S = x.shape
    V, D = tok_table.shape
    C = 8
    idx = x.reshape(-1).astype(jnp.int32)
    pe = jnp.asarray(_PE_NP[:S], dtype=jnp.float32)
    out = _build(B, S, D, C)(idx, tok_table, pe)
    return out.reshape(B, S, D)
